# baseline (speedup 1.0000x reference)
import jax
import jax.numpy as jnp
from jax import lax
from jax.experimental import pallas as pl
from jax.experimental.pallas import tpu as pltpu
from jax.experimental.pallas import tpu_sc as plsc

NC = 2
NS = 16
NW = NC * NS

N_NODES = 100000
MEM_DIM = 500
B = 16384
BPW = B // NW
GCH = 64
NCHUNK = BPW // GCH

RW = 3128
RWP = 3136
LOGB = 14
CCH = 80

_mesh = plsc.VectorSubcoreMesh(core_axis_name="c", subcore_axis_name="s",
                               num_cores=NC, num_subcores=NS)


def _iota16():
    return jnp.arange(16, dtype=jnp.int32)


def _vgather16(x, sel):
    dnums = lax.GatherDimensionNumbers(
        offset_dims=(), collapsed_slice_dims=(0,), start_index_map=(0,))
    return lax.gather(x, sel[:, None], dnums, (1,),
                      mode=lax.GatherScatterMode.PROMISE_IN_BOUNDS)


def _worker_id():
    return lax.axis_index("s") * NC + lax.axis_index("c")


def _plan(ids_v, claim_v, wsrc_v, wdst_v, lo, hi):
    iota = _iota16()

    def initbody(g, x):
        claim_v[pl.ds(g * 16, 16)] = jnp.full((16,), -1, jnp.int32)
        return x
    lax.fori_loop(0, RWP // 16, initbody, 0)

    shifted_sel = jnp.minimum(iota + 1, 15)

    def scanbody(t, x):
        n = ids_v[pl.ds(t * 16, 16)]
        key = (n << LOGB) | (t * 16 + iota)
        srt = lax.sort(key)
        n_s = srt >> LOGB
        i_s = srt & (B - 1)
        nxt = _vgather16(n_s, shifted_sel)
        win = (iota == 15) | (nxt != n_s)
        m = win & (n_s >= lo) & (n_s < hi)
        plsc.store_scatter(claim_v, [n_s - lo], i_s, mask=m)
        return x
    lax.fori_loop(0, B // 16, scanbody, 0)

    def compactbody(g, off):
        v = claim_v[pl.ds(g * 16, 16)]
        m = v >= 0
        plsc.store_compressed(wsrc_v.at[pl.ds(off, 16)], v, mask=m)
        plsc.store_compressed(wdst_v.at[pl.ds(off, 16)], lo + g * 16 + iota,
                              mask=m)
        return off + jnp.sum(jnp.where(m, 1, 0))
    return lax.fori_loop(0, RWP // 16, compactbody, jnp.int32(0))


def _gather_body(ids_hbm, mem_hbm, lu_hbm,
                 h_out, lu_out, wsrc_out, wdst_out, cnt_out,
                 ids_v, hbuf, lubuf, claim_v, wsrc_v, wdst_v, cnt_v,
                 sem1, sem2):
    wid = _worker_id()
    base = wid * BPW
    lo = wid * RW
    hi = jnp.minimum(lo + RW, N_NODES)
    pltpu.sync_copy(ids_hbm, ids_v)

    for k in range(NCHUNK):
        row = ids_v.at[pl.ds(base + k * GCH, GCH)]
        cp2 = pltpu.async_copy(lu_hbm.at[row], lubuf, sem2)
        for q in range(GCH // 16):
            v = ids_v[pl.ds(base + k * GCH + q * 16, 16)]
            for j in range(16):
                pltpu.make_async_copy(
                    mem_hbm.at[pl.ds(v[j], 1)],
                    hbuf.at[pl.ds(q * 16 + j, 1)],
                    sem1,
                ).start()
        pltpu.make_async_copy(mem_hbm.at[pl.ds(0, GCH)], hbuf, sem1).wait()
        pltpu.sync_copy(hbuf, h_out.at[pl.ds(base + k * GCH, GCH)])
        cp2.wait()
        pltpu.sync_copy(lubuf, lu_out.at[pl.ds(base + k * GCH, GCH)])

    cnt = _plan(ids_v, claim_v, wsrc_v, wdst_v, lo, hi)
    cnt_v[...] = jnp.full((16,), cnt, jnp.int32)
    pltpu.sync_copy(wsrc_v, wsrc_out.at[wid])
    pltpu.sync_copy(wdst_v, wdst_out.at[wid])
    pltpu.sync_copy(cnt_v, cnt_out.at[wid])


_gather_call = pl.kernel(
    _gather_body,
    out_type=[
        jax.ShapeDtypeStruct((B, MEM_DIM), jnp.float32),
        jax.ShapeDtypeStruct((B,), jnp.float32),
        jax.ShapeDtypeStruct((NW, RWP), jnp.int32),
        jax.ShapeDtypeStruct((NW, RWP), jnp.int32),
        jax.ShapeDtypeStruct((NW, 16), jnp.int32),
    ],
    mesh=_mesh,
    scratch_types=[
        pltpu.VMEM((B,), jnp.int32),
        pltpu.VMEM((GCH, MEM_DIM), jnp.float32),
        pltpu.VMEM((GCH,), jnp.float32),
        pltpu.VMEM((RWP,), jnp.int32),
        pltpu.VMEM((RWP,), jnp.int32),
        pltpu.VMEM((RWP,), jnp.int32),
        pltpu.VMEM((16,), jnp.int32),
        pltpu.SemaphoreType.DMA,
        pltpu.SemaphoreType.DMA,
    ],
    compiler_params=pltpu.CompilerParams(needs_layout_passes=False),
)


def _scatter_body(mem_hbm, hnew_hbm, wsrc_hbm, wdst_hbm, cnt_hbm, out_hbm,
                  wsrc_v, wdst_v, cnt_v, cbuf, rowbuf, semr, semw, semg, sems):
    wid = _worker_id()
    lo = wid * RW
    rw = jnp.minimum(lo + RW, N_NODES) - lo
    iota = _iota16()

    pltpu.sync_copy(wsrc_hbm.at[wid], wsrc_v)
    pltpu.sync_copy(wdst_hbm.at[wid], wdst_v)
    pltpu.sync_copy(cnt_hbm.at[wid], cnt_v)
    cnt = cnt_v[pl.ds(0, 16)][0]

    nf = rw // CCH

    def off_of(t):
        return pl.multiple_of(
            jnp.where(t >= nf, lo + rw - CCH, lo + t * CCH), 8)

    def rd(t):
        o = off_of(t)
        pltpu.make_async_copy(mem_hbm.at[pl.ds(o, CCH)],
                              cbuf.at[t % 2], semr).start()

    def drain_rd():
        pltpu.make_async_copy(mem_hbm.at[pl.ds(0, CCH)],
                              cbuf.at[0], semr).wait()

    def wr(t):
        o = off_of(t)
        pltpu.make_async_copy(cbuf.at[t % 2],
                              out_hbm.at[pl.ds(o, CCH)], semw).start()

    def drain_wr():
        pltpu.make_async_copy(cbuf.at[0],
                              out_hbm.at[pl.ds(0, CCH)], semw).wait()

    rd(0)

    def cpbody(t, x):
        drain_rd()
        @pl.when(t >= 1)
        def _():
            drain_wr()
        wr(t)
        rd(t + 1)
        return x
    lax.fori_loop(0, nf + 1, cpbody, 0)
    drain_rd()
    drain_wr()

    def chunk(src16, dst16):
        for j in range(16):
            pltpu.make_async_copy(hnew_hbm.at[pl.ds(src16[j], 1)],
                                  rowbuf.at[pl.ds(j, 1)], semg).start()
        pltpu.make_async_copy(hnew_hbm.at[pl.ds(0, 16)], rowbuf, semg).wait()
        for j in range(16):
            pltpu.make_async_copy(rowbuf.at[pl.ds(j, 1)],
                                  out_hbm.at[pl.ds(dst16[j], 1)], sems).start()
        pltpu.make_async_copy(rowbuf, out_hbm.at[pl.ds(0, 16)], sems).wait()

    def scatbody(t, x):
        chunk(wsrc_v[pl.ds(t * 16, 16)], wdst_v[pl.ds(t * 16, 16)])
        return x
    lax.fori_loop(0, cnt >> 4, scatbody, 0)

    @pl.when(cnt > 0)
    def _():
        sel = jnp.maximum(iota + cnt - 16, 0)
        chunk(plsc.load_gather(wsrc_v, [sel]),
              plsc.load_gather(wdst_v, [sel]))


_scatter_call = pl.kernel(
    _scatter_body,
    out_type=jax.ShapeDtypeStruct((N_NODES, MEM_DIM), jnp.float32),
    mesh=_mesh,
    scratch_types=[
        pltpu.VMEM((RWP,), jnp.int32),
        pltpu.VMEM((RWP,), jnp.int32),
        pltpu.VMEM((16,), jnp.int32),
        pltpu.VMEM((2, CCH, MEM_DIM), jnp.float32),
        pltpu.VMEM((16, MEM_DIM), jnp.float32),
        pltpu.SemaphoreType.DMA,
        pltpu.SemaphoreType.DMA,
        pltpu.SemaphoreType.DMA,
        pltpu.SemaphoreType.DMA,
    ],
    compiler_params=pltpu.CompilerParams(needs_layout_passes=False),
)


BB = 512
NBB = B // BB


def _tc_body(h, ef, et, lu, wt, bt, w1a, w1b, w1c, b1, w2, b2,
             wxr, wxz, wxn, whr, whz, whn, bxr, bxz, bxn, bhr, bhz, bhn,
             hn_out):
    f32 = jnp.float32
    bf16 = jnp.bfloat16
    hh = h[...]
    td = et[...] - lu[...]
    te = jnp.cos(td * wt[...] + bt[...])
    hb = hh.astype(bf16)

    def bdot(a, w):
        return jnp.dot(a.astype(bf16), w[...].astype(bf16),
                       preferred_element_type=f32)

    hid = (bdot(hh, w1a) + bdot(ef[...], w1b) + bdot(te, w1c) + b1[...])
    hid = jnp.maximum(hid, 0.0)
    msg = bdot(hid, w2) + b2[...]
    xr = bdot(msg, wxr) + bxr[...]
    xz = bdot(msg, wxz) + bxz[...]
    xn = bdot(msg, wxn) + bxn[...]
    hr = jnp.dot(hb, whr[...].astype(bf16), preferred_element_type=f32) + bhr[...]
    hz = jnp.dot(hb, whz[...].astype(bf16), preferred_element_type=f32) + bhz[...]
    hn = jnp.dot(hb, whn[...].astype(bf16), preferred_element_type=f32) + bhn[...]
    r = jax.nn.sigmoid(xr + hr)
    z = jax.nn.sigmoid(xz + hz)
    n = jnp.tanh(xn + r * hn)
    hn_out[...] = (1.0 - z) * n + z * hh


def _const_spec(shape):
    nd = len(shape)
    return pl.BlockSpec(shape, lambda i: (0,) * nd)


def _tc_compute(h, ef, et1, lu1, wt2, bt2, w1a, w1b, w1c, b1_2, w2, b2_2,
                gw, gb):
    in_specs = [
        pl.BlockSpec((BB, MEM_DIM), lambda i: (i, 0)),
        pl.BlockSpec((BB, ef.shape[1]), lambda i: (i, 0)),
        pl.BlockSpec((BB, 1), lambda i: (i, 0)),
        pl.BlockSpec((BB, 1), lambda i: (i, 0)),
        _const_spec(wt2.shape),
        _const_spec(bt2.shape),
        _const_spec(w1a.shape),
        _const_spec(w1b.shape),
        _const_spec(w1c.shape),
        _const_spec(b1_2.shape),
        _const_spec(w2.shape),
        _const_spec(b2_2.shape),
    ] + [_const_spec(w.shape) for w in gw] + [_const_spec(b.shape) for b in gb]
    return pl.pallas_call(
        _tc_body,
        grid=(NBB,),
        in_specs=in_specs,
        out_specs=pl.BlockSpec((BB, MEM_DIM), lambda i: (i, 0)),
        out_shape=jax.ShapeDtypeStruct((B, MEM_DIM), jnp.float32),
    )(h, ef, et1, lu1, wt2, bt2, w1a, w1b, w1c, b1_2, w2, b2_2, *gw, *gb)


def kernel(memory, last_update, node_ids, edge_feats, edge_times,
           w_t, b_t, W1, b1, W2, b2, W_ih, b_ih, W_hh, b_hh):
    ids = node_ids.astype(jnp.int32)
    h, lu, wsrc, wdst, cnts = _gather_call(ids, memory, last_update)

    w1a = W1[:MEM_DIM]
    w1b = W1[MEM_DIM:MEM_DIM + 17]
    w1c = W1[MEM_DIM + 17:]
    gw = [W_ih[:, :MEM_DIM], W_ih[:, MEM_DIM:2 * MEM_DIM], W_ih[:, 2 * MEM_DIM:],
          W_hh[:, :MEM_DIM], W_hh[:, MEM_DIM:2 * MEM_DIM], W_hh[:, 2 * MEM_DIM:]]
    gb = [b_ih[:MEM_DIM][None], b_ih[MEM_DIM:2 * MEM_DIM][None], b_ih[2 * MEM_DIM:][None],
          b_hh[:MEM_DIM][None], b_hh[MEM_DIM:2 * MEM_DIM][None], b_hh[2 * MEM_DIM:][None]]
    h_new = _tc_compute(h, edge_feats, edge_times[:, None],
                        lu[:, None], w_t[None], b_t[None],
                        w1a, w1b, w1c, b1[None], W2, b2[None], gw, gb)

    return _scatter_call(memory, h_new, wsrc, wdst, cnts)


# speedup vs baseline: 1.2451x; 1.2451x over previous
import jax
import jax.numpy as jnp
from jax import lax
from jax.experimental import pallas as pl
from jax.experimental.pallas import tpu as pltpu
from jax.experimental.pallas import tpu_sc as plsc

NC = 2
NS = 16
NW = NC * NS

N_NODES = 100000
MEM_DIM = 500
B = 16384
BPW = B // NW
GCH = 64
NCHUNK = BPW // GCH

RW = 3128
RWP = 3136
LOGB = 14
CCH = 80

_mesh = plsc.VectorSubcoreMesh(core_axis_name="c", subcore_axis_name="s",
                               num_cores=NC, num_subcores=NS)


def _iota16():
    return jnp.arange(16, dtype=jnp.int32)


def _vgather16(x, sel):
    dnums = lax.GatherDimensionNumbers(
        offset_dims=(), collapsed_slice_dims=(0,), start_index_map=(0,))
    return lax.gather(x, sel[:, None], dnums, (1,),
                      mode=lax.GatherScatterMode.PROMISE_IN_BOUNDS)


def _worker_id():
    return lax.axis_index("s") * NC + lax.axis_index("c")


def _plan(ids_v, claim_v, wsrc_v, wdst_v, lo, hi):
    iota = _iota16()

    def initbody(g, x):
        claim_v[pl.ds(g * 16, 16)] = jnp.full((16,), -1, jnp.int32)
        return x
    lax.fori_loop(0, RWP // 16, initbody, 0)

    shifted_sel = jnp.minimum(iota + 1, 15)

    def scanbody(t, x):
        n = ids_v[pl.ds(t * 16, 16)]
        key = (n << LOGB) | (t * 16 + iota)
        srt = lax.sort(key)
        n_s = srt >> LOGB
        i_s = srt & (B - 1)
        nxt = _vgather16(n_s, shifted_sel)
        win = (iota == 15) | (nxt != n_s)
        m = win & (n_s >= lo) & (n_s < hi)
        plsc.store_scatter(claim_v, [n_s - lo], i_s, mask=m)
        return x
    lax.fori_loop(0, B // 16, scanbody, 0)

    def compactbody(g, off):
        v = claim_v[pl.ds(g * 16, 16)]
        m = v >= 0
        plsc.store_compressed(wsrc_v.at[pl.ds(off, 16)], v, mask=m)
        plsc.store_compressed(wdst_v.at[pl.ds(off, 16)], lo + g * 16 + iota,
                              mask=m)
        return off + jnp.sum(jnp.where(m, 1, 0))
    return lax.fori_loop(0, RWP // 16, compactbody, jnp.int32(0))


def _gather_body(ids_hbm, mem_hbm, lu_hbm,
                 h_out, lu_out, wsrc_out, wdst_out, cnt_out,
                 ids_v, hbuf, lubuf, claim_v, wsrc_v, wdst_v, cnt_v,
                 sem1, sem2):
    wid = _worker_id()
    base = wid * BPW
    lo = wid * RW
    hi = jnp.minimum(lo + RW, N_NODES)
    pltpu.sync_copy(ids_hbm, ids_v)

    for k in range(NCHUNK):
        row = ids_v.at[pl.ds(base + k * GCH, GCH)]
        cp2 = pltpu.async_copy(lu_hbm.at[row], lubuf, sem2)
        for q in range(GCH // 16):
            v = ids_v[pl.ds(base + k * GCH + q * 16, 16)]
            for j in range(16):
                pltpu.make_async_copy(
                    mem_hbm.at[pl.ds(v[j], 1)],
                    hbuf.at[pl.ds(q * 16 + j, 1)],
                    sem1,
                ).start()
        pltpu.make_async_copy(mem_hbm.at[pl.ds(0, GCH)], hbuf, sem1).wait()
        pltpu.sync_copy(hbuf, h_out.at[pl.ds(base + k * GCH, GCH)])
        cp2.wait()
        pltpu.sync_copy(lubuf, lu_out.at[pl.ds(base + k * GCH, GCH)])

    cnt = _plan(ids_v, claim_v, wsrc_v, wdst_v, lo, hi)
    cnt_v[...] = jnp.full((16,), cnt, jnp.int32)
    pltpu.sync_copy(wsrc_v, wsrc_out.at[wid])
    pltpu.sync_copy(wdst_v, wdst_out.at[wid])
    pltpu.sync_copy(cnt_v, cnt_out.at[wid])


_gather_call = pl.kernel(
    _gather_body,
    out_type=[
        jax.ShapeDtypeStruct((B, MEM_DIM), jnp.float32),
        jax.ShapeDtypeStruct((B,), jnp.float32),
        jax.ShapeDtypeStruct((NW, RWP), jnp.int32),
        jax.ShapeDtypeStruct((NW, RWP), jnp.int32),
        jax.ShapeDtypeStruct((NW, 16), jnp.int32),
    ],
    mesh=_mesh,
    scratch_types=[
        pltpu.VMEM((B,), jnp.int32),
        pltpu.VMEM((GCH, MEM_DIM), jnp.float32),
        pltpu.VMEM((GCH,), jnp.float32),
        pltpu.VMEM((RWP,), jnp.int32),
        pltpu.VMEM((RWP,), jnp.int32),
        pltpu.VMEM((RWP,), jnp.int32),
        pltpu.VMEM((16,), jnp.int32),
        pltpu.SemaphoreType.DMA,
        pltpu.SemaphoreType.DMA,
    ],
    compiler_params=pltpu.CompilerParams(needs_layout_passes=False),
)


def _scatter_body(hnew_hbm, wsrc_hbm, wdst_hbm, cnt_hbm, out_hbm,
                  wsrc_v, wdst_v, cnt_v, rowbuf, semg, sems):
    wid = _worker_id()
    iota = _iota16()

    pltpu.sync_copy(wsrc_hbm.at[wid], wsrc_v)
    pltpu.sync_copy(wdst_hbm.at[wid], wdst_v)
    pltpu.sync_copy(cnt_hbm.at[wid], cnt_v)
    cnt = cnt_v[pl.ds(0, 16)][0]

    def chunk(src16, dst16):
        for j in range(16):
            pltpu.make_async_copy(hnew_hbm.at[pl.ds(src16[j], 1)],
                                  rowbuf.at[pl.ds(j, 1)], semg).start()
        pltpu.make_async_copy(hnew_hbm.at[pl.ds(0, 16)], rowbuf, semg).wait()
        for j in range(16):
            pltpu.make_async_copy(rowbuf.at[pl.ds(j, 1)],
                                  out_hbm.at[pl.ds(dst16[j], 1)], sems).start()
        pltpu.make_async_copy(rowbuf, out_hbm.at[pl.ds(0, 16)], sems).wait()

    def scatbody(t, x):
        chunk(wsrc_v[pl.ds(t * 16, 16)], wdst_v[pl.ds(t * 16, 16)])
        return x
    lax.fori_loop(0, cnt >> 4, scatbody, 0)

    @pl.when(cnt > 0)
    def _():
        sel = jnp.maximum(iota + cnt - 16, 0)
        chunk(plsc.load_gather(wsrc_v, [sel]),
              plsc.load_gather(wdst_v, [sel]))


_scatter_call = pl.kernel(
    _scatter_body,
    out_type=(),
    mesh=_mesh,
    scratch_types=[
        pltpu.VMEM((RWP,), jnp.int32),
        pltpu.VMEM((RWP,), jnp.int32),
        pltpu.VMEM((16,), jnp.int32),
        pltpu.VMEM((16, MEM_DIM), jnp.float32),
        pltpu.SemaphoreType.DMA,
        pltpu.SemaphoreType.DMA,
    ],
    compiler_params=pltpu.CompilerParams(needs_layout_passes=False),
)


BB = 512
NBB = B // BB


def _tc_body(h, ef, et, lu, wt, bt, w1a, w1b, w1c, b1, w2, b2,
             wxr, wxz, wxn, whr, whz, whn, bxr, bxz, bxn, bhr, bhz, bhn,
             hn_out):
    f32 = jnp.float32
    bf16 = jnp.bfloat16
    hh = h[...]
    td = et[...] - lu[...]
    te = jnp.cos(td * wt[...] + bt[...])
    hb = hh.astype(bf16)

    def bdot(a, w):
        return jnp.dot(a.astype(bf16), w[...].astype(bf16),
                       preferred_element_type=f32)

    hid = (bdot(hh, w1a) + bdot(ef[...], w1b) + bdot(te, w1c) + b1[...])
    hid = jnp.maximum(hid, 0.0)
    msg = bdot(hid, w2) + b2[...]
    xr = bdot(msg, wxr) + bxr[...]
    xz = bdot(msg, wxz) + bxz[...]
    xn = bdot(msg, wxn) + bxn[...]
    hr = jnp.dot(hb, whr[...].astype(bf16), preferred_element_type=f32) + bhr[...]
    hz = jnp.dot(hb, whz[...].astype(bf16), preferred_element_type=f32) + bhz[...]
    hn = jnp.dot(hb, whn[...].astype(bf16), preferred_element_type=f32) + bhn[...]
    r = jax.nn.sigmoid(xr + hr)
    z = jax.nn.sigmoid(xz + hz)
    n = jnp.tanh(xn + r * hn)
    hn_out[...] = (1.0 - z) * n + z * hh


def _const_spec(shape):
    nd = len(shape)
    return pl.BlockSpec(shape, lambda i: (0,) * nd)


def _tc_compute(h, ef, et1, lu1, wt2, bt2, w1a, w1b, w1c, b1_2, w2, b2_2,
                gw, gb):
    in_specs = [
        pl.BlockSpec((BB, MEM_DIM), lambda i: (i, 0)),
        pl.BlockSpec((BB, ef.shape[1]), lambda i: (i, 0)),
        pl.BlockSpec((BB, 1), lambda i: (i, 0)),
        pl.BlockSpec((BB, 1), lambda i: (i, 0)),
        _const_spec(wt2.shape),
        _const_spec(bt2.shape),
        _const_spec(w1a.shape),
        _const_spec(w1b.shape),
        _const_spec(w1c.shape),
        _const_spec(b1_2.shape),
        _const_spec(w2.shape),
        _const_spec(b2_2.shape),
    ] + [_const_spec(w.shape) for w in gw] + [_const_spec(b.shape) for b in gb]
    return pl.pallas_call(
        _tc_body,
        grid=(NBB,),
        in_specs=in_specs,
        out_specs=pl.BlockSpec((BB, MEM_DIM), lambda i: (i, 0)),
        out_shape=jax.ShapeDtypeStruct((B, MEM_DIM), jnp.float32),
    )(h, ef, et1, lu1, wt2, bt2, w1a, w1b, w1c, b1_2, w2, b2_2, *gw, *gb)


def kernel(memory, last_update, node_ids, edge_feats, edge_times,
           w_t, b_t, W1, b1, W2, b2, W_ih, b_ih, W_hh, b_hh):
    ids = node_ids.astype(jnp.int32)
    h, lu, wsrc, wdst, cnts = _gather_call(ids, memory, last_update)

    w1a = W1[:MEM_DIM]
    w1b = W1[MEM_DIM:MEM_DIM + 17]
    w1c = W1[MEM_DIM + 17:]
    gw = [W_ih[:, :MEM_DIM], W_ih[:, MEM_DIM:2 * MEM_DIM], W_ih[:, 2 * MEM_DIM:],
          W_hh[:, :MEM_DIM], W_hh[:, MEM_DIM:2 * MEM_DIM], W_hh[:, 2 * MEM_DIM:]]
    gb = [b_ih[:MEM_DIM][None], b_ih[MEM_DIM:2 * MEM_DIM][None], b_ih[2 * MEM_DIM:][None],
          b_hh[:MEM_DIM][None], b_hh[MEM_DIM:2 * MEM_DIM][None], b_hh[2 * MEM_DIM:][None]]
    h_new = _tc_compute(h, edge_feats, edge_times[:, None],
                        lu[:, None], w_t[None], b_t[None],
                        w1a, w1b, w1c, b1[None], W2, b2[None], gw, gb)

    ref = jax.new_ref(memory)
    _scatter_call(h_new, wsrc, wdst, cnts, ref)
    return jax.freeze(ref)


# speedup vs baseline: 1.2766x; 1.0253x over previous
import jax
import jax.numpy as jnp
from jax import lax
from jax.experimental import pallas as pl
from jax.experimental.pallas import tpu as pltpu
from jax.experimental.pallas import tpu_sc as plsc

NC = 2
NS = 16
NW = NC * NS

N_NODES = 100000
MEM_DIM = 500
B = 16384
BPW = B // NW
GCH = 64
NCHUNK = BPW // GCH

RW = 3128
RWP = 3136
LOGB = 14
CCH = 80

_mesh = plsc.VectorSubcoreMesh(core_axis_name="c", subcore_axis_name="s",
                               num_cores=NC, num_subcores=NS)


def _iota16():
    return jnp.arange(16, dtype=jnp.int32)


def _vgather16(x, sel):
    dnums = lax.GatherDimensionNumbers(
        offset_dims=(), collapsed_slice_dims=(0,), start_index_map=(0,))
    return lax.gather(x, sel[:, None], dnums, (1,),
                      mode=lax.GatherScatterMode.PROMISE_IN_BOUNDS)


def _worker_id():
    return lax.axis_index("s") * NC + lax.axis_index("c")


def _plan(ids_v, claim_v, wsrc_v, wdst_v, lo, hi):
    iota = _iota16()

    def initbody(g, x):
        claim_v[pl.ds(g * 16, 16)] = jnp.full((16,), -1, jnp.int32)
        return x
    lax.fori_loop(0, RWP // 16, initbody, 0)

    shifted_sel = jnp.minimum(iota + 1, 15)

    def scanbody(t, x):
        n = ids_v[pl.ds(t * 16, 16)]
        key = (n << LOGB) | (t * 16 + iota)
        srt = lax.sort(key)
        n_s = srt >> LOGB
        i_s = srt & (B - 1)
        nxt = _vgather16(n_s, shifted_sel)
        win = (iota == 15) | (nxt != n_s)
        m = win & (n_s >= lo) & (n_s < hi)
        plsc.store_scatter(claim_v, [n_s - lo], i_s, mask=m)
        return x
    lax.fori_loop(0, B // 16, scanbody, 0)

    def compactbody(g, off):
        v = claim_v[pl.ds(g * 16, 16)]
        m = v >= 0
        plsc.store_compressed(wsrc_v.at[pl.ds(off, 16)], v, mask=m)
        plsc.store_compressed(wdst_v.at[pl.ds(off, 16)], lo + g * 16 + iota,
                              mask=m)
        return off + jnp.sum(jnp.where(m, 1, 0))
    return lax.fori_loop(0, RWP // 16, compactbody, jnp.int32(0))


def _gather_body(ids_hbm, mem_hbm, lu_hbm,
                 h_out, lu_out, wsrc_out, wdst_out, cnt_out,
                 ids_v, hbuf, lubuf, claim_v, wsrc_v, wdst_v, cnt_v,
                 sem1, sem2):
    wid = _worker_id()
    base = wid * BPW
    lo = wid * RW
    hi = jnp.minimum(lo + RW, N_NODES)
    pltpu.sync_copy(ids_hbm, ids_v)

    for k in range(NCHUNK):
        row = ids_v.at[pl.ds(base + k * GCH, GCH)]
        cp2 = pltpu.async_copy(lu_hbm.at[row], lubuf, sem2)
        for q in range(GCH // 16):
            v = ids_v[pl.ds(base + k * GCH + q * 16, 16)]
            for j in range(16):
                pltpu.make_async_copy(
                    mem_hbm.at[pl.ds(v[j], 1)],
                    hbuf.at[pl.ds(q * 16 + j, 1)],
                    sem1,
                ).start()
        pltpu.make_async_copy(mem_hbm.at[pl.ds(0, GCH)], hbuf, sem1).wait()
        pltpu.sync_copy(hbuf, h_out.at[pl.ds(base + k * GCH, GCH)])
        cp2.wait()
        pltpu.sync_copy(lubuf, lu_out.at[pl.ds(base + k * GCH, GCH)])

    cnt = _plan(ids_v, claim_v, wsrc_v, wdst_v, lo, hi)
    cnt_v[...] = jnp.full((16,), cnt, jnp.int32)
    pltpu.sync_copy(wsrc_v, wsrc_out.at[wid])
    pltpu.sync_copy(wdst_v, wdst_out.at[wid])
    pltpu.sync_copy(cnt_v, cnt_out.at[wid])


_gather_call = pl.kernel(
    _gather_body,
    out_type=[
        jax.ShapeDtypeStruct((B, MEM_DIM), jnp.float32),
        jax.ShapeDtypeStruct((B,), jnp.float32),
        jax.ShapeDtypeStruct((NW, RWP), jnp.int32),
        jax.ShapeDtypeStruct((NW, RWP), jnp.int32),
        jax.ShapeDtypeStruct((NW, 16), jnp.int32),
    ],
    mesh=_mesh,
    scratch_types=[
        pltpu.VMEM((B,), jnp.int32),
        pltpu.VMEM((GCH, MEM_DIM), jnp.float32),
        pltpu.VMEM((GCH,), jnp.float32),
        pltpu.VMEM((RWP,), jnp.int32),
        pltpu.VMEM((RWP,), jnp.int32),
        pltpu.VMEM((RWP,), jnp.int32),
        pltpu.VMEM((16,), jnp.int32),
        pltpu.SemaphoreType.DMA,
        pltpu.SemaphoreType.DMA,
    ],
    compiler_params=pltpu.CompilerParams(needs_layout_passes=False),
)


def _scatter_body(hnew_hbm, wsrc_hbm, wdst_hbm, cnt_hbm, out_hbm,
                  wsrc_v, wdst_v, cnt_v, rowbuf, semg, sems):
    wid = _worker_id()
    iota = _iota16()

    pltpu.sync_copy(wsrc_hbm.at[wid], wsrc_v)
    pltpu.sync_copy(wdst_hbm.at[wid], wdst_v)
    pltpu.sync_copy(cnt_hbm.at[wid], cnt_v)
    cnt = cnt_v[pl.ds(0, 16)][0]

    def chunk(src16, dst16):
        for j in range(16):
            pltpu.make_async_copy(hnew_hbm.at[pl.ds(src16[j], 1)],
                                  rowbuf.at[pl.ds(j, 1)], semg).start()
        pltpu.make_async_copy(hnew_hbm.at[pl.ds(0, 16)], rowbuf, semg).wait()
        for j in range(16):
            pltpu.make_async_copy(rowbuf.at[pl.ds(j, 1)],
                                  out_hbm.at[pl.ds(dst16[j], 1)], sems).start()
        pltpu.make_async_copy(rowbuf, out_hbm.at[pl.ds(0, 16)], sems).wait()

    def scatbody(t, x):
        chunk(wsrc_v[pl.ds(t * 16, 16)], wdst_v[pl.ds(t * 16, 16)])
        return x
    lax.fori_loop(0, cnt >> 4, scatbody, 0)

    @pl.when(cnt > 0)
    def _():
        sel = jnp.maximum(iota + cnt - 16, 0)
        chunk(plsc.load_gather(wsrc_v, [sel]),
              plsc.load_gather(wdst_v, [sel]))


_scatter_call = pl.kernel(
    _scatter_body,
    out_type=(),
    mesh=_mesh,
    scratch_types=[
        pltpu.VMEM((RWP,), jnp.int32),
        pltpu.VMEM((RWP,), jnp.int32),
        pltpu.VMEM((16,), jnp.int32),
        pltpu.VMEM((16, MEM_DIM), jnp.float32),
        pltpu.SemaphoreType.DMA,
        pltpu.SemaphoreType.DMA,
    ],
    compiler_params=pltpu.CompilerParams(needs_layout_passes=False),
)


BB = 1024
NBB = B // BB


def _tc_body(h, ef, et, lu, wt, bt, w1a, w1b, w1c, b1, w2, b2,
             wxr, wxz, wxn, whr, whz, whn, bxr, bxz, bxn, bhr, bhz, bhn,
             hn_out):
    f32 = jnp.float32
    bf16 = jnp.bfloat16
    hh = h[...]
    td = et[...] - lu[...]
    te = jnp.cos(td * wt[...] + bt[...])
    hb = hh.astype(bf16)

    def bdot(a, w):
        return jnp.dot(a.astype(bf16), w[...].astype(bf16),
                       preferred_element_type=f32)

    hid = (bdot(hh, w1a) + bdot(ef[...], w1b) + bdot(te, w1c) + b1[...])
    hid = jnp.maximum(hid, 0.0)
    msg = bdot(hid, w2) + b2[...]
    xr = bdot(msg, wxr) + bxr[...]
    xz = bdot(msg, wxz) + bxz[...]
    xn = bdot(msg, wxn) + bxn[...]
    hr = jnp.dot(hb, whr[...].astype(bf16), preferred_element_type=f32) + bhr[...]
    hz = jnp.dot(hb, whz[...].astype(bf16), preferred_element_type=f32) + bhz[...]
    hn = jnp.dot(hb, whn[...].astype(bf16), preferred_element_type=f32) + bhn[...]
    r = jax.nn.sigmoid(xr + hr)
    z = jax.nn.sigmoid(xz + hz)
    n = jnp.tanh(xn + r * hn)
    hn_out[...] = (1.0 - z) * n + z * hh


def _const_spec(shape):
    nd = len(shape)
    return pl.BlockSpec(shape, lambda i: (0,) * nd)


def _tc_compute(h, ef, et1, lu1, wt2, bt2, w1a, w1b, w1c, b1_2, w2, b2_2,
                gw, gb):
    in_specs = [
        pl.BlockSpec((BB, MEM_DIM), lambda i: (i, 0)),
        pl.BlockSpec((BB, ef.shape[1]), lambda i: (i, 0)),
        pl.BlockSpec((BB, 1), lambda i: (i, 0)),
        pl.BlockSpec((BB, 1), lambda i: (i, 0)),
        _const_spec(wt2.shape),
        _const_spec(bt2.shape),
        _const_spec(w1a.shape),
        _const_spec(w1b.shape),
        _const_spec(w1c.shape),
        _const_spec(b1_2.shape),
        _const_spec(w2.shape),
        _const_spec(b2_2.shape),
    ] + [_const_spec(w.shape) for w in gw] + [_const_spec(b.shape) for b in gb]
    return pl.pallas_call(
        _tc_body,
        grid=(NBB,),
        in_specs=in_specs,
        out_specs=pl.BlockSpec((BB, MEM_DIM), lambda i: (i, 0)),
        out_shape=jax.ShapeDtypeStruct((B, MEM_DIM), jnp.float32),
    )(h, ef, et1, lu1, wt2, bt2, w1a, w1b, w1c, b1_2, w2, b2_2, *gw, *gb)


def kernel(memory, last_update, node_ids, edge_feats, edge_times,
           w_t, b_t, W1, b1, W2, b2, W_ih, b_ih, W_hh, b_hh):
    ids = node_ids.astype(jnp.int32)
    h, lu, wsrc, wdst, cnts = _gather_call(ids, memory, last_update)

    w1a = W1[:MEM_DIM]
    w1b = W1[MEM_DIM:MEM_DIM + 17]
    w1c = W1[MEM_DIM + 17:]
    gw = [W_ih[:, :MEM_DIM], W_ih[:, MEM_DIM:2 * MEM_DIM], W_ih[:, 2 * MEM_DIM:],
          W_hh[:, :MEM_DIM], W_hh[:, MEM_DIM:2 * MEM_DIM], W_hh[:, 2 * MEM_DIM:]]
    gb = [b_ih[:MEM_DIM][None], b_ih[MEM_DIM:2 * MEM_DIM][None], b_ih[2 * MEM_DIM:][None],
          b_hh[:MEM_DIM][None], b_hh[MEM_DIM:2 * MEM_DIM][None], b_hh[2 * MEM_DIM:][None]]
    h_new = _tc_compute(h, edge_feats, edge_times[:, None],
                        lu[:, None], w_t[None], b_t[None],
                        w1a, w1b, w1c, b1[None], W2, b2[None], gw, gb)

    ref = jax.new_ref(memory)
    _scatter_call(h_new, wsrc, wdst, cnts, ref)
    return jax.freeze(ref)
